# baseline (speedup 1.0000x reference)
import jax
import jax.numpy as jnp
from jax import lax
from jax.experimental import pallas as pl
from jax.experimental.pallas import tpu as pltpu

_DEG = 3
_OUT = 256
_NCP = 32
_KL = 36
_KP = 128


def _axis_matrix(kv_ref, n_out):
    f32 = jnp.float32
    i32 = jnp.int32

    iota_k = lax.broadcasted_iota(i32, (1, _KP), 1)
    valid = iota_k < _KL
    kv_raw = kv_ref[...]
    kcl = jnp.where(kv_raw < 0.0, 0.0001, kv_raw)
    kcl = jnp.where(valid, kcl, 0.0)

    kc = kcl
    for s in (1, 2, 4, 8, 16, 32):
        kc = kc + pltpu.roll(kc, s, 1)

    k0 = kc[:, 0:1]
    klast = kc[:, _KL - 1 : _KL]
    kvn = (kc - k0) / (klast - k0)

    step = (1.0 - 2e-05) / (n_out - 1)
    ep = (
        lax.broadcasted_iota(i32, (n_out, 1), 0).astype(f32) * step + 1e-05
    )

    iota2 = lax.broadcasted_iota(i32, (n_out, _KP), 1)
    diff = ep - kvn
    in_band = (iota2 >= _DEG) & (iota2 < _KL - 2 * _DEG + _DEG)
    masked = jnp.where(diff > 1e-08, diff, 1.0)
    masked = jnp.where(in_band, masked, 2.0)
    minv = jnp.min(masked, axis=1, keepdims=True)
    cand = jnp.where(masked == minv, iota2, _KP + 1)
    span = jnp.min(cand, axis=1, keepdims=True)

    def kv_at(offset):
        oh = (iota2 == span + offset).astype(f32)
        return jnp.sum(oh * kvn, axis=1, keepdims=True)

    kv_off = {o: kv_at(o) for o in range(-2, 4)}

    basis = [jnp.zeros((n_out, 1), f32) for _ in range(_DEG + 1)]
    basis[0] = jnp.ones((n_out, 1), f32)
    for k in range(1, _DEG + 1):
        saved = jnp.zeros((n_out, 1), f32)
        for r in range(k):
            left = kv_off[r + 1]
            right = kv_off[1 - k + r]
            denom = (left - ep) + (ep - right)
            temp = basis[r] / denom
            temp = jnp.where(denom == 0.0, 0.0001, temp)
            basis[r] = saved + (left - ep) * temp
            saved = (ep - right) * temp
        basis[k] = saved

    iota_c = lax.broadcasted_iota(i32, (n_out, _NCP), 1)
    amat = jnp.zeros((n_out, _NCP), f32)
    for l in range(_DEG + 1):
        t0 = span - (_DEG + l)
        tgt = jnp.where(t0 < 0, t0 + _NCP, t0)
        amat = amat + jnp.where(iota_c == tgt, basis[l], 0.0)
    return amat


def _body(cp_ref, kvx_ref, kvy_ref, out_ref):
    ax = _axis_matrix(kvx_ref, _OUT)
    ay = _axis_matrix(kvy_ref, _OUT)
    for d in range(3):
        tmp = jnp.dot(ax, cp_ref[d], preferred_element_type=jnp.float32, precision=lax.Precision.HIGHEST)
        out_ref[d] = lax.dot_general(
            tmp, ay, (((1,), (1,)), ((), ())),
            preferred_element_type=jnp.float32,
            precision=lax.Precision.HIGHEST,
        )


def kernel(control_points, knot_vector_x, knot_vector_y):
    cp = jnp.transpose(control_points, (2, 0, 1))
    kvx = jnp.pad(knot_vector_x, ((0, 0), (0, _KP - _KL)))
    kvy = jnp.pad(knot_vector_y, ((0, 0), (0, _KP - _KL)))
    out = pl.pallas_call(
        _body,
        out_shape=jax.ShapeDtypeStruct((3, _OUT, _OUT), jnp.float32),
    )(cp, kvx, kvy)
    return jnp.transpose(out, (1, 2, 0))[None]


# speedup vs baseline: 1.3960x; 1.3960x over previous
import jax
import jax.numpy as jnp
from jax import lax
from jax.experimental import pallas as pl
from jax.experimental.pallas import tpu as pltpu

_DEG = 3
_OUT = 256
_NCP = 32
_KL = 36
_KP = 128


def _axis_matrix(kv_ref, n_out):
    f32 = jnp.float32
    i32 = jnp.int32

    kv_raw = kv_ref[...]
    kcl = jnp.where(kv_raw < 0.0, 0.0001, kv_raw)
    kcl = jnp.concatenate(
        [kcl, jnp.zeros((1, _KP - _KL), f32)], axis=1)

    kc = kcl
    for s in (1, 2, 4, 8, 16, 32):
        kc = kc + pltpu.roll(kc, s, 1)

    k0 = kc[:, 0:1]
    klast = kc[:, _KL - 1 : _KL]
    kvn = (kc - k0) / (klast - k0)

    step = (1.0 - 2e-05) / (n_out - 1)
    ep = (
        lax.broadcasted_iota(i32, (n_out, 1), 0).astype(f32) * step + 1e-05
    )

    iota2 = lax.broadcasted_iota(i32, (n_out, _KP), 1)
    diff = ep - kvn
    in_band = (iota2 >= _DEG) & (iota2 < _KL - 2 * _DEG + _DEG)
    masked = jnp.where(diff > 1e-08, diff, 1.0)
    masked = jnp.where(in_band, masked, 2.0)
    minv = jnp.min(masked, axis=1, keepdims=True)
    cand = jnp.where(masked == minv, iota2, _KP + 1)
    span = jnp.min(cand, axis=1, keepdims=True)

    def kv_at(offset):
        oh = (iota2 == span + offset).astype(f32)
        return jnp.sum(oh * kvn, axis=1, keepdims=True)

    kv_off = {o: kv_at(o) for o in range(-2, 4)}

    basis = [jnp.zeros((n_out, 1), f32) for _ in range(_DEG + 1)]
    basis[0] = jnp.ones((n_out, 1), f32)
    for k in range(1, _DEG + 1):
        saved = jnp.zeros((n_out, 1), f32)
        for r in range(k):
            left = kv_off[r + 1]
            right = kv_off[1 - k + r]
            denom = (left - ep) + (ep - right)
            temp = basis[r] / denom
            temp = jnp.where(denom == 0.0, 0.0001, temp)
            basis[r] = saved + (left - ep) * temp
            saved = (ep - right) * temp
        basis[k] = saved

    iota_c = lax.broadcasted_iota(i32, (n_out, _NCP), 1)
    amat = jnp.zeros((n_out, _NCP), f32)
    for l in range(_DEG + 1):
        t0 = span - (_DEG + l)
        tgt = jnp.where(t0 < 0, t0 + _NCP, t0)
        amat = amat + jnp.where(iota_c == tgt, basis[l], 0.0)
    return amat


def _body(cp_ref, kvx_ref, kvy_ref, out_ref):
    ax = _axis_matrix(kvx_ref, _OUT)
    ay = _axis_matrix(kvy_ref, _OUT)
    for d in range(3):
        tmp = jnp.dot(ax, cp_ref[d], preferred_element_type=jnp.float32, precision=lax.Precision.HIGHEST)
        out_ref[d] = lax.dot_general(
            tmp, ay, (((1,), (1,)), ((), ())),
            preferred_element_type=jnp.float32,
            precision=lax.Precision.HIGHEST,
        )


def kernel(control_points, knot_vector_x, knot_vector_y):
    cp = jnp.transpose(control_points, (2, 0, 1))
    kvx = knot_vector_x
    kvy = knot_vector_y
    out = pl.pallas_call(
        _body,
        out_shape=jax.ShapeDtypeStruct((3, _OUT, _OUT), jnp.float32),
    )(cp, kvx, kvy)
    return jnp.transpose(out, (1, 2, 0))[None]


# speedup vs baseline: 1.6047x; 1.1495x over previous
import jax
import jax.numpy as jnp
from jax import lax
from jax.experimental import pallas as pl
from jax.experimental.pallas import tpu as pltpu

_DEG = 3
_OUT = 256
_NCP = 32
_KL = 36
_KP = 128
_KS = 40


def _axis_matrix_t(kv_ref, n_out):
    f32 = jnp.float32
    i32 = jnp.int32

    kv_raw = kv_ref[...]
    kcl = jnp.where(kv_raw < 0.0, 0.0001, kv_raw)
    kcl = jnp.concatenate(
        [kcl, jnp.zeros((1, _KP - _KL), f32)], axis=1)

    kc = kcl
    for s in (1, 2, 4, 8, 16, 32):
        kc = kc + pltpu.roll(kc, s, 1)

    k0 = kc[:, 0:1]
    klast = kc[:, _KL - 1 : _KL]
    kvn = (kc - k0) / (klast - k0)

    ones_row = jnp.ones((1, n_out), f32)
    kvb = lax.dot_general(
        kvn, ones_row, (((0,), (0,)), ((), ())),
        preferred_element_type=f32, precision=lax.Precision.HIGHEST,
    )[:_KS]

    step = (1.0 - 2e-05) / (n_out - 1)
    ep = (
        lax.broadcasted_iota(i32, (1, n_out), 1).astype(f32) * step + 1e-05
    )

    riota = lax.broadcasted_iota(i32, (_KS, n_out), 0)
    in_band = (riota >= _DEG) & (riota < _KL - _DEG)
    diff = ep - kvb
    pos = diff > 1e-08
    masked = jnp.where(pos, diff, 1.0)
    masked = jnp.where(in_band, masked, 2.0)
    minv = jnp.min(masked, axis=0, keepdims=True)
    cnt = jnp.sum((pos & in_band).astype(i32), axis=0, keepdims=True)
    cgt = jnp.sum((masked > minv).astype(i32), axis=0, keepdims=True)
    span = jnp.where(cnt > 0, _DEG + cgt - _KS + cnt, _DEG)

    def kv_at(offset):
        oh = (riota == span + offset).astype(f32)
        return jnp.sum(oh * kvb, axis=0, keepdims=True)

    kv_off = {o: kv_at(o) for o in range(-2, 4)}

    basis = [jnp.zeros((1, n_out), f32) for _ in range(_DEG + 1)]
    basis[0] = jnp.ones((1, n_out), f32)
    for k in range(1, _DEG + 1):
        saved = jnp.zeros((1, n_out), f32)
        for r in range(k):
            left = kv_off[r + 1]
            right = kv_off[1 - k + r]
            denom = (left - ep) + (ep - right)
            temp = basis[r] / denom
            temp = jnp.where(denom == 0.0, 0.0001, temp)
            basis[r] = saved + (left - ep) * temp
            saved = (ep - right) * temp
        basis[k] = saved

    ciota = lax.broadcasted_iota(i32, (_NCP, n_out), 0)
    amat_t = jnp.zeros((_NCP, n_out), f32)
    for l in range(_DEG + 1):
        t0 = span - (_DEG + l)
        tgt = jnp.where(t0 < 0, t0 + _NCP, t0)
        amat_t = amat_t + jnp.where(ciota == tgt, basis[l], 0.0)
    return amat_t


def _body(cp_ref, kvx_ref, kvy_ref, out_ref):
    axt = _axis_matrix_t(kvx_ref, _OUT)
    ayt = _axis_matrix_t(kvy_ref, _OUT)
    tmps = []
    for d in range(3):
        tmps.append(lax.dot_general(
            axt, cp_ref[d], (((0,), (0,)), ((), ())),
            preferred_element_type=jnp.float32,
            precision=lax.Precision.HIGHEST,
        ))
    stacked = jnp.concatenate(tmps, axis=0)
    out = jnp.dot(stacked, ayt, preferred_element_type=jnp.float32,
                  precision=lax.Precision.HIGHEST)
    out_ref[...] = jnp.reshape(out, (3, _OUT, _OUT))


def kernel(control_points, knot_vector_x, knot_vector_y):
    cp = jnp.transpose(control_points, (2, 0, 1))
    out = pl.pallas_call(
        _body,
        out_shape=jax.ShapeDtypeStruct((3, _OUT, _OUT), jnp.float32),
    )(cp, knot_vector_x, knot_vector_y)
    return jnp.transpose(out, (1, 2, 0))[None]


# speedup vs baseline: 1.6264x; 1.0135x over previous
import jax
import jax.numpy as jnp
from jax import lax
from jax.experimental import pallas as pl
from jax.experimental.pallas import tpu as pltpu

_DEG = 3
_OUT = 256
_NCP = 32
_KL = 36
_KP = 128
_KS = 40


def _axis_matrix_t(kv_ref, n_out):
    f32 = jnp.float32
    i32 = jnp.int32

    kv_raw = kv_ref[...]
    kcl = jnp.where(kv_raw < 0.0, 0.0001, kv_raw)
    kcl = jnp.concatenate(
        [kcl, jnp.zeros((1, _KP - _KL), f32)], axis=1)

    kc = kcl
    for s in (1, 2, 4, 8, 16, 32):
        kc = kc + pltpu.roll(kc, s, 1)

    k0 = kc[:, 0:1]
    klast = kc[:, _KL - 1 : _KL]
    kvn = (kc - k0) / (klast - k0)

    ones_row = jnp.ones((1, n_out), f32)
    kvb = lax.dot_general(
        kvn[:, :_KS], ones_row, (((0,), (0,)), ((), ())),
        preferred_element_type=f32, precision=lax.Precision.HIGHEST,
    )

    step = (1.0 - 2e-05) / (n_out - 1)
    ep = (
        lax.broadcasted_iota(i32, (1, n_out), 1).astype(f32) * step + 1e-05
    )

    riota = lax.broadcasted_iota(i32, (_KS, n_out), 0)
    in_band = (riota >= _DEG) & (riota < _KL - _DEG)
    diff = ep - kvb
    pos = diff > 1e-08
    masked = jnp.where(pos, diff, 1.0)
    masked = jnp.where(in_band, masked, 2.0)
    minv = jnp.min(masked, axis=0, keepdims=True)
    cnt = jnp.sum((pos & in_band).astype(i32), axis=0, keepdims=True)
    cgt = jnp.sum((masked > minv).astype(i32), axis=0, keepdims=True)
    span = jnp.where(cnt > 0, _DEG + cgt - _KS + cnt, _DEG)

    def kv_at(offset):
        oh = (riota == span + offset).astype(f32)
        return jnp.sum(oh * kvb, axis=0, keepdims=True)

    kv_off = {o: kv_at(o) for o in range(-2, 4)}

    basis = [jnp.zeros((1, n_out), f32) for _ in range(_DEG + 1)]
    basis[0] = jnp.ones((1, n_out), f32)
    for k in range(1, _DEG + 1):
        saved = jnp.zeros((1, n_out), f32)
        for r in range(k):
            left = kv_off[r + 1]
            right = kv_off[1 - k + r]
            denom = (left - ep) + (ep - right)
            temp = basis[r] / denom
            temp = jnp.where(denom == 0.0, 0.0001, temp)
            basis[r] = saved + (left - ep) * temp
            saved = (ep - right) * temp
        basis[k] = saved

    ciota = lax.broadcasted_iota(i32, (_NCP, n_out), 0)
    amat_t = jnp.zeros((_NCP, n_out), f32)
    for l in range(_DEG + 1):
        t0 = span - (_DEG + l)
        tgt = jnp.where(t0 < 0, t0 + _NCP, t0)
        amat_t = amat_t + jnp.where(ciota == tgt, basis[l], 0.0)
    return amat_t


def _body(cp_ref, kvx_ref, kvy_ref, out_ref):
    axt = _axis_matrix_t(kvx_ref, _OUT)
    ayt = _axis_matrix_t(kvy_ref, _OUT)
    tmps = []
    for d in range(3):
        tmps.append(lax.dot_general(
            axt, cp_ref[d], (((0,), (0,)), ((), ())),
            preferred_element_type=jnp.float32,
            precision=lax.Precision.HIGHEST,
        ))
    stacked = jnp.concatenate(tmps, axis=0)
    out = jnp.dot(stacked, ayt, preferred_element_type=jnp.float32,
                  precision=lax.Precision.HIGHEST)
    out_ref[...] = jnp.reshape(out, (3, _OUT, _OUT))


def kernel(control_points, knot_vector_x, knot_vector_y):
    cp = jnp.transpose(control_points, (2, 0, 1))
    out = pl.pallas_call(
        _body,
        out_shape=jax.ShapeDtypeStruct((3, _OUT, _OUT), jnp.float32),
    )(cp, knot_vector_x, knot_vector_y)
    return jnp.transpose(out, (1, 2, 0))[None]


# speedup vs baseline: 1.6583x; 1.0196x over previous
import jax
import jax.numpy as jnp
from jax import lax
from jax.experimental import pallas as pl
from jax.experimental.pallas import tpu as pltpu

_DEG = 3
_OUT = 256
_NCP = 32
_KL = 36
_KP = 128
_KS = 40


def _axis_matrix_t(kv_ref, n_out):
    f32 = jnp.float32
    i32 = jnp.int32

    kv_raw = kv_ref[...]
    kcl = jnp.where(kv_raw < 0.0, 0.0001, kv_raw)
    kcl = jnp.concatenate(
        [kcl, jnp.zeros((1, _KP - _KL), f32)], axis=1)

    kc = kcl
    for s in (1, 2, 4, 8, 16, 32):
        kc = kc + pltpu.roll(kc, s, 1)

    k0 = kc[:, 0:1]
    klast = kc[:, _KL - 1 : _KL]
    kvn = (kc - k0) / (klast - k0)

    ones_row = jnp.ones((1, n_out), f32)
    kvb = lax.dot_general(
        kvn[:, :_KS], ones_row, (((0,), (0,)), ((), ())),
        preferred_element_type=f32, precision=lax.Precision.HIGHEST,
    )

    step = (1.0 - 2e-05) / (n_out - 1)
    ep = (
        lax.broadcasted_iota(i32, (1, n_out), 1).astype(f32) * step + 1e-05
    )

    riota = lax.broadcasted_iota(i32, (_KS, n_out), 0)
    in_band = (riota >= _DEG) & (riota < _KL - _DEG)
    diff = ep - kvb
    pos = diff > 1e-08
    masked = jnp.where(pos, diff, 1.0)
    masked = jnp.where(in_band, masked, 2.0)
    minv = jnp.min(masked, axis=0, keepdims=True)
    cnt = jnp.sum((pos & in_band).astype(i32), axis=0, keepdims=True)
    cgt = jnp.sum((masked > minv).astype(i32), axis=0, keepdims=True)
    span = jnp.where(cnt > 0, _DEG + cgt - _KS + cnt, _DEG)

    def kv_at(offset):
        oh = (riota == span + offset).astype(f32)
        return jnp.sum(oh * kvb, axis=0, keepdims=True)

    kv_off = {o: kv_at(o) for o in range(-2, 4)}

    basis = [jnp.zeros((1, n_out), f32) for _ in range(_DEG + 1)]
    basis[0] = jnp.ones((1, n_out), f32)
    for k in range(1, _DEG + 1):
        saved = jnp.zeros((1, n_out), f32)
        for r in range(k):
            left = kv_off[r + 1]
            right = kv_off[1 - k + r]
            denom = (left - ep) + (ep - right)
            temp = basis[r] / denom
            temp = jnp.where(denom == 0.0, 0.0001, temp)
            basis[r] = saved + (left - ep) * temp
            saved = (ep - right) * temp
        basis[k] = saved

    ciota = lax.broadcasted_iota(i32, (_NCP, n_out), 0)
    amat_t = jnp.zeros((_NCP, n_out), f32)
    for l in range(_DEG + 1):
        t0 = span - (_DEG + l)
        tgt = jnp.where(t0 < 0, t0 + _NCP, t0)
        amat_t = amat_t + jnp.where(ciota == tgt, basis[l], 0.0)
    return amat_t


def _body(cp_hbm, kvx_ref, kvy_ref, out_ref, cp_vmem, sem):
    copy = pltpu.make_async_copy(cp_hbm, cp_vmem, sem)
    copy.start()
    axt = _axis_matrix_t(kvx_ref, _OUT)
    ayt = _axis_matrix_t(kvy_ref, _OUT)
    copy.wait()
    tmps = []
    for d in range(3):
        tmps.append(lax.dot_general(
            axt, cp_vmem[d], (((0,), (0,)), ((), ())),
            preferred_element_type=jnp.float32,
            precision=lax.Precision.HIGHEST,
        ))
    stacked = jnp.concatenate(tmps, axis=0)
    out = jnp.dot(stacked, ayt, preferred_element_type=jnp.float32,
                  precision=lax.Precision.HIGHEST)
    out_ref[...] = jnp.reshape(out, (3, _OUT, _OUT))


def kernel(control_points, knot_vector_x, knot_vector_y):
    cp = jnp.transpose(control_points, (2, 0, 1))
    out = pl.pallas_call(
        _body,
        out_shape=jax.ShapeDtypeStruct((3, _OUT, _OUT), jnp.float32),
        in_specs=[
            pl.BlockSpec(memory_space=pltpu.MemorySpace.HBM),
            pl.BlockSpec(memory_space=pltpu.MemorySpace.VMEM),
            pl.BlockSpec(memory_space=pltpu.MemorySpace.VMEM),
        ],
        scratch_shapes=[
            pltpu.VMEM((3, _NCP, _NCP), jnp.float32),
            pltpu.SemaphoreType.DMA,
        ],
    )(cp, knot_vector_x, knot_vector_y)
    return jnp.transpose(out, (1, 2, 0))[None]
